# baseline (speedup 1.0000x reference)
import jax
import jax.numpy as jnp
from jax.experimental import pallas as pl

BLOCK_S = 256


def _add_kernel(x_ref, pos_ref, o_ref):
    o_ref[...] = x_ref[...] + pos_ref[...]


def kernel(x, pos_table):
    batch, seq_len, hidden = x.shape
    grid = (seq_len // BLOCK_S,)
    return pl.pallas_call(
        _add_kernel,
        grid=grid,
        in_specs=[
            pl.BlockSpec((batch, BLOCK_S, hidden), lambda s: (0, s, 0)),
            pl.BlockSpec((BLOCK_S, hidden), lambda s: (s, 0)),
        ],
        out_specs=pl.BlockSpec((batch, BLOCK_S, hidden), lambda s: (0, s, 0)),
        out_shape=jax.ShapeDtypeStruct(x.shape, x.dtype),
    )(x, pos_table)


# speedup vs baseline: 1.0069x; 1.0069x over previous
import jax
import jax.numpy as jnp
from jax.experimental import pallas as pl

BLOCK_S = 512


def _add_kernel(x_ref, pos_ref, o_ref):
    o_ref[...] = x_ref[...] + pos_ref[...]


def kernel(x, pos_table):
    batch, seq_len, hidden = x.shape
    grid = (seq_len // BLOCK_S,)
    return pl.pallas_call(
        _add_kernel,
        grid=grid,
        in_specs=[
            pl.BlockSpec((batch, BLOCK_S, hidden), lambda s: (0, s, 0)),
            pl.BlockSpec((BLOCK_S, hidden), lambda s: (s, 0)),
        ],
        out_specs=pl.BlockSpec((batch, BLOCK_S, hidden), lambda s: (0, s, 0)),
        out_shape=jax.ShapeDtypeStruct(x.shape, x.dtype),
    )(x, pos_table)


# speedup vs baseline: 1.0075x; 1.0005x over previous
import jax
import jax.numpy as jnp
from jax.experimental import pallas as pl
from jax.experimental.pallas import tpu as pltpu

CHUNK = 512
NBUF = 8
LA = 4
PCH = 16


def _body(x_hbm, pos_hbm, o_hbm, pos_vmem, bufs, in_sems, out_sems, pos_sems):
    rows = x_hbm.shape[0]
    nch = rows // CHUNK

    def in_copy(j):
        return pltpu.make_async_copy(
            x_hbm.at[pl.ds(j * CHUNK, CHUNK), :],
            bufs.at[j % NBUF],
            in_sems.at[j % NBUF],
        )

    def out_copy(j):
        return pltpu.make_async_copy(
            bufs.at[j % NBUF],
            o_hbm.at[pl.ds(j * CHUNK, CHUNK), :],
            out_sems.at[j % NBUF],
        )

    def pos_copy(k):
        return pltpu.make_async_copy(
            pos_hbm.at[pl.ds(k * CHUNK, CHUNK), :],
            pos_vmem.at[pl.ds(k * CHUNK, CHUNK), :],
            pos_sems.at[k],
        )

    for j in range(min(LA, nch)):
        in_copy(j).start()
    for k in range(PCH):
        pos_copy(k).start()

    last_out_waited = 0
    for i in range(nch):
        in_copy(i).wait()
        if i < PCH:
            pos_copy(i).wait()
        p = (i % PCH) * CHUNK
        s = i % NBUF
        bufs[s] = bufs[s] + pos_vmem[pl.ds(p, CHUNK), :]
        out_copy(i).start()
        j = i + LA
        if j < nch:
            if j - NBUF >= 0:
                out_copy(j - NBUF).wait()
                last_out_waited = j - NBUF + 1
            in_copy(j).start()
    for k in range(last_out_waited, nch):
        out_copy(k).wait()


def kernel(x, pos_table):
    batch, seq_len, hidden = x.shape
    xr = x.reshape(batch * seq_len, hidden)
    out = pl.pallas_call(
        _body,
        in_specs=[
            pl.BlockSpec(memory_space=pltpu.MemorySpace.HBM),
            pl.BlockSpec(memory_space=pltpu.MemorySpace.HBM),
        ],
        out_specs=pl.BlockSpec(memory_space=pltpu.MemorySpace.HBM),
        out_shape=jax.ShapeDtypeStruct(xr.shape, x.dtype),
        scratch_shapes=[
            pltpu.VMEM((seq_len, hidden), jnp.float32),
            pltpu.VMEM((NBUF, CHUNK, hidden), jnp.float32),
            pltpu.SemaphoreType.DMA((NBUF,)),
            pltpu.SemaphoreType.DMA((NBUF,)),
            pltpu.SemaphoreType.DMA((PCH,)),
        ],
    )(xr, pos_table)
    return out.reshape(batch, seq_len, hidden)
